# Initial kernel scaffold; baseline (speedup 1.0000x reference)
import jax
import jax.numpy as jnp
from jax.experimental import pallas as pl


def kernel(input_ids, params):
    raise NotImplementedError("write your pallas kernel here")


# speedup vs baseline: 1.2878x; 1.2878x over previous
import functools

import jax
import jax.numpy as jnp
import numpy as np
from jax.experimental import pallas as pl
from jax.experimental.pallas import tpu as pltpu

D_MODEL = 768
N_HEADS = 12
HEAD_DIM = D_MODEL // N_HEADS
D_FF = 1024
N_EXPERTS = 8
TOP_K = 2
VOCAB = 8192
MAX_LEN = 2048

ROW_BLK = 256
HI = jax.lax.Precision.HIGHEST


def _ln(x, g, b):
    m = jnp.mean(x, axis=-1, keepdims=True)
    v = jnp.mean((x - m) * (x - m), axis=-1, keepdims=True)
    return (x - m) / jnp.sqrt(v + 1e-5) * g + b


def _bf(t):
    return t.astype(jnp.bfloat16)


def _embed_krn(ids_ref, tok_ref, pos_ref, o_ref):
    def body(j, _):
        t = ids_ref[j]
        o_ref[pl.ds(j, 1), :] = tok_ref[pl.ds(t, 1), :] + pos_ref[pl.ds(j, 1), :]
        return 0

    jax.lax.fori_loop(0, o_ref.shape[0], body, 0, unroll=8)


def _embed(ids, tok_emb, pos_emb, L):
    return pl.pallas_call(
        _embed_krn,
        grid_spec=pltpu.PrefetchScalarGridSpec(
            num_scalar_prefetch=1,
            grid=(1,),
            in_specs=[
                pl.BlockSpec((VOCAB, D_MODEL), lambda i, ids: (0, 0)),
                pl.BlockSpec((L, D_MODEL), lambda i, ids: (0, 0)),
            ],
            out_specs=pl.BlockSpec((L, D_MODEL), lambda i, ids: (0, 0)),
        ),
        out_shape=jax.ShapeDtypeStruct((L, D_MODEL), jnp.float32),
    )(ids, tok_emb, pos_emb)


def _ln_mm_krn(x_ref, g_ref, b_ref, w_ref, bias_ref, o_ref):
    xn = _ln(x_ref[...], g_ref[0], b_ref[0])
    o_ref[...] = (
        jax.lax.dot_general(
            _bf(xn), _bf(w_ref[...]), (((1,), (1,)), ((), ())),
            preferred_element_type=jnp.float32,
        )
        + bias_ref[0]
    )


def _ln_mm(x, g, b, w, bias, L):
    OUT = w.shape[0]
    nblk = L // ROW_BLK
    return pl.pallas_call(
        _ln_mm_krn,
        grid=(nblk,),
        in_specs=[
            pl.BlockSpec((ROW_BLK, D_MODEL), lambda i: (i, 0)),
            pl.BlockSpec((1, D_MODEL), lambda i: (0, 0)),
            pl.BlockSpec((1, D_MODEL), lambda i: (0, 0)),
            pl.BlockSpec((OUT, D_MODEL), lambda i: (0, 0)),
            pl.BlockSpec((1, OUT), lambda i: (0, 0)),
        ],
        out_specs=pl.BlockSpec((ROW_BLK, OUT), lambda i: (i, 0)),
        out_shape=jax.ShapeDtypeStruct((L, OUT), jnp.float32),
    )(x, g.reshape(1, -1), b.reshape(1, -1), w, bias.reshape(1, -1))


KV_BLK = 1024


def _attn_krn(q_ref, k_ref, v_ref, o_ref):
    q = q_ref[0]
    QB = q.shape[0]
    L = k_ref.shape[1]
    m = jnp.full((QB, 1), -jnp.inf, jnp.float32)
    ssum = jnp.zeros((QB, 1), jnp.float32)
    o = jnp.zeros((QB, HEAD_DIM), jnp.float32)
    for j in range(L // KV_BLK):
        k_j = k_ref[0, j * KV_BLK:(j + 1) * KV_BLK]
        v_j = v_ref[0, j * KV_BLK:(j + 1) * KV_BLK]
        s = jax.lax.dot_general(
            _bf(q), _bf(k_j), (((1,), (1,)), ((), ())),
            preferred_element_type=jnp.float32,
        ) / np.sqrt(HEAD_DIM).astype(np.float32)
        m_blk = jnp.max(s, axis=-1, keepdims=True)
        m_new = jnp.maximum(m, m_blk)
        delta = jnp.where(m == m_new, 0.0, m - m_new)
        p = jnp.exp(s - m_new)
        bs = jnp.sum(p, axis=-1, keepdims=True)
        scale = jnp.exp(delta) * ssum
        ssum = scale + bs
        o = jnp.dot(_bf(p), _bf(v_j), preferred_element_type=jnp.float32) \
            + scale * o
        o = o * (1.0 / ssum)
        m = m_new
    o_ref[0] = o


def _attention(q, k, v, L):
    qblk = 256
    return pl.pallas_call(
        _attn_krn,
        grid=(N_HEADS, L // qblk),
        in_specs=[
            pl.BlockSpec((1, qblk, HEAD_DIM), lambda h, i: (h, i, 0)),
            pl.BlockSpec((1, L, HEAD_DIM), lambda h, i: (h, 0, 0)),
            pl.BlockSpec((1, L, HEAD_DIM), lambda h, i: (h, 0, 0)),
        ],
        out_specs=pl.BlockSpec((1, qblk, HEAD_DIM), lambda h, i: (h, i, 0)),
        out_shape=jax.ShapeDtypeStruct((N_HEADS, L, HEAD_DIM), jnp.float32),
    )(q, k, v)


def _mm_res_krn(x_ref, w_ref, bias_ref, res_ref, o_ref):
    o_ref[...] = (
        jax.lax.dot_general(
            _bf(x_ref[...]), _bf(w_ref[...]), (((1,), (1,)), ((), ())),
            preferred_element_type=jnp.float32,
        )
        + bias_ref[0]
        + res_ref[...]
    )


def _mm_res(x, w, bias, res, L):
    nblk = L // ROW_BLK
    return pl.pallas_call(
        _mm_res_krn,
        grid=(nblk,),
        in_specs=[
            pl.BlockSpec((ROW_BLK, D_MODEL), lambda i: (i, 0)),
            pl.BlockSpec((D_MODEL, D_MODEL), lambda i: (0, 0)),
            pl.BlockSpec((1, D_MODEL), lambda i: (0, 0)),
            pl.BlockSpec((ROW_BLK, D_MODEL), lambda i: (i, 0)),
        ],
        out_specs=pl.BlockSpec((ROW_BLK, D_MODEL), lambda i: (i, 0)),
        out_shape=jax.ShapeDtypeStruct((L, D_MODEL), jnp.float32),
    )(x, w, bias.reshape(1, -1), res)


T_DISP = 128
N_ASSIGN = 2 * MAX_LEN
MAXT = N_ASSIGN // T_DISP + N_EXPERTS


def _router_krn(x_ref, g_ref, b_ref, gw_ref, xn_ref, e1_ref, e2_ref, wa_ref, wb_ref):
    x = x_ref[...]
    xn = _ln(x, g_ref[0], b_ref[0])
    xn_ref[...] = xn
    logits = jax.lax.dot_general(
        _bf(xn), _bf(gw_ref[...]), (((1,), (1,)), ((), ())),
        preferred_element_type=jnp.float32,
    )
    ids8 = jax.lax.broadcasted_iota(jnp.int32, logits.shape, 1)
    m1 = jnp.max(logits, axis=-1, keepdims=True)
    i1 = jnp.min(jnp.where(logits == m1, ids8, N_EXPERTS), axis=-1, keepdims=True)
    masked = jnp.where(ids8 == i1, -jnp.inf, logits)
    m2 = jnp.max(masked, axis=-1, keepdims=True)
    i2 = jnp.min(jnp.where(masked == m2, ids8, N_EXPERTS), axis=-1, keepdims=True)
    s = jnp.exp(m2 - m1)
    wa = 1.0 / (1.0 + s)
    e1_ref[...] = i1
    e2_ref[...] = i2
    wa_ref[...] = wa
    wb_ref[...] = 1.0 - wa


def _router(x, g, b, gate_w, L):
    nblk = L // ROW_BLK
    return pl.pallas_call(
        _router_krn,
        grid=(nblk,),
        in_specs=[
            pl.BlockSpec((ROW_BLK, D_MODEL), lambda i: (i, 0)),
            pl.BlockSpec((1, D_MODEL), lambda i: (0, 0)),
            pl.BlockSpec((1, D_MODEL), lambda i: (0, 0)),
            pl.BlockSpec((N_EXPERTS, D_MODEL), lambda i: (0, 0)),
        ],
        out_specs=[
            pl.BlockSpec((ROW_BLK, D_MODEL), lambda i: (i, 0)),
            pl.BlockSpec((ROW_BLK, 1), lambda i: (i, 0)),
            pl.BlockSpec((ROW_BLK, 1), lambda i: (i, 0)),
            pl.BlockSpec((ROW_BLK, 1), lambda i: (i, 0)),
            pl.BlockSpec((ROW_BLK, 1), lambda i: (i, 0)),
        ],
        out_shape=[
            jax.ShapeDtypeStruct((L, D_MODEL), jnp.float32),
            jax.ShapeDtypeStruct((L, 1), jnp.int32),
            jax.ShapeDtypeStruct((L, 1), jnp.int32),
            jax.ShapeDtypeStruct((L, 1), jnp.float32),
            jax.ShapeDtypeStruct((L, 1), jnp.float32),
        ],
    )(x, g.reshape(1, -1), b.reshape(1, -1), gate_w)


def _rank_krn(e_ref, rank_ref, cnt_ref, run_ref):
    i = pl.program_id(0)

    @pl.when(i == 0)
    def _():
        run_ref[...] = jnp.zeros_like(run_ref)

    e = e_ref[...]
    iota8 = jax.lax.broadcasted_iota(jnp.int32, (e.shape[0], N_EXPERTS), 1)
    onehot = (e == iota8).astype(jnp.float32)
    r = jax.lax.broadcasted_iota(jnp.int32, (e.shape[0], e.shape[0]), 0)
    c = jax.lax.broadcasted_iota(jnp.int32, (e.shape[0], e.shape[0]), 1)
    tri = (r > c).astype(jnp.float32)
    rank_in_blk = jnp.dot(tri, onehot, preferred_element_type=jnp.float32,
                          precision=HI)
    rank = jnp.sum(onehot * (rank_in_blk + run_ref[...]), axis=-1, keepdims=True)
    rank_ref[...] = rank.astype(jnp.int32)
    run_ref[...] += jnp.sum(onehot, axis=0, keepdims=True)

    @pl.when(i == pl.num_programs(0) - 1)
    def _():
        cnt_ref[...] = run_ref[...].astype(jnp.int32)


def _ranks(e_all, NA):
    nblk = NA // ROW_BLK
    return pl.pallas_call(
        _rank_krn,
        grid=(nblk,),
        in_specs=[pl.BlockSpec((ROW_BLK, 1), lambda i: (i, 0))],
        out_specs=[
            pl.BlockSpec((ROW_BLK, 1), lambda i: (i, 0)),
            pl.BlockSpec((1, N_EXPERTS), lambda i: (0, 0)),
        ],
        out_shape=[
            jax.ShapeDtypeStruct((NA, 1), jnp.int32),
            jax.ShapeDtypeStruct((1, N_EXPERTS), jnp.int32),
        ],
        scratch_shapes=[pltpu.VMEM((1, N_EXPERTS), jnp.float32)],
    )(e_all)


def _pos_krn(cnt_ref, rank_ref, e_ref, pos_ref, te_ref, nt_ref):
    NA = rank_ref.shape[0]
    cnt = cnt_ref[...].astype(jnp.float32)
    ntiles = jnp.floor((cnt + (T_DISP - 1)) * (1.0 / T_DISP))
    r = jax.lax.broadcasted_iota(jnp.int32, (N_EXPERTS, N_EXPERTS), 0)
    c = jax.lax.broadcasted_iota(jnp.int32, (N_EXPERTS, N_EXPERTS), 1)
    mat = (r < c).astype(jnp.float32)
    tile_start = jnp.dot(ntiles, mat, preferred_element_type=jnp.float32,
                         precision=HI)
    off = tile_start * T_DISP

    e = e_ref[...]
    iota8 = jax.lax.broadcasted_iota(jnp.int32, (NA, N_EXPERTS), 1)
    onehot = (e == iota8).astype(jnp.float32)
    off_e = jnp.sum(onehot * off, axis=-1, keepdims=True)
    pos_ref[...] = off_e.astype(jnp.int32) + rank_ref[...]

    tt = jax.lax.broadcasted_iota(jnp.int32, (1, MAXT), 1).astype(jnp.float32)
    te = jnp.zeros((1, MAXT), jnp.float32)
    for ee in range(N_EXPERTS):
        lo = tile_start[0, ee]
        hi = lo + ntiles[0, ee]
        te = te + jnp.where((tt >= lo) & (tt < hi), float(ee), 0.0)
    te_ref[...] = te.astype(jnp.int32)
    nt_ref[...] = jnp.sum(ntiles, axis=-1, keepdims=True).astype(jnp.int32)


def _positions(cnt, rank, e_all, NA):
    return pl.pallas_call(
        _pos_krn,
        grid=(1,),
        in_specs=[
            pl.BlockSpec((1, N_EXPERTS), lambda i: (0, 0)),
            pl.BlockSpec((NA, 1), lambda i: (0, 0)),
            pl.BlockSpec((NA, 1), lambda i: (0, 0)),
        ],
        out_specs=[
            pl.BlockSpec((NA, 1), lambda i: (0, 0)),
            pl.BlockSpec((1, MAXT), lambda i: (0, 0)),
            pl.BlockSpec((1, 1), lambda i: (0, 0)),
        ],
        out_shape=[
            jax.ShapeDtypeStruct((NA, 1), jnp.int32),
            jax.ShapeDtypeStruct((1, MAXT), jnp.int32),
            jax.ShapeDtypeStruct((1, 1), jnp.int32),
        ],
    )(cnt, rank, e_all)


def _disp_krn(pos_ref, xn_ref, w_ref, xd_ref, wd_ref):
    L = xn_ref.shape[0]

    def body(j, _):
        p = pos_ref[j]
        xd_ref[pl.ds(p, 1), :] = xn_ref[pl.ds(j, 1), :]
        wd_ref[pl.ds(p, 1), :] = w_ref[pl.ds(j, 1), :]
        return 0

    jax.lax.fori_loop(0, L, body, 0, unroll=8)

    def body2(j, _):
        p = pos_ref[L + j]
        xd_ref[pl.ds(p, 1), :] = xn_ref[pl.ds(j, 1), :]
        wd_ref[pl.ds(p, 1), :] = w_ref[pl.ds(L + j, 1), :]
        return 0

    jax.lax.fori_loop(0, L, body2, 0, unroll=8)


def _dispatch(pos, xn, w_all, L):
    NA = 2 * L
    ND = MAXT * T_DISP
    return pl.pallas_call(
        _disp_krn,
        grid_spec=pltpu.PrefetchScalarGridSpec(
            num_scalar_prefetch=1,
            grid=(1,),
            in_specs=[
                pl.BlockSpec((L, D_MODEL), lambda i, pos: (0, 0)),
                pl.BlockSpec((NA, 1), lambda i, pos: (0, 0)),
            ],
            out_specs=[
                pl.BlockSpec((ND, D_MODEL), lambda i, pos: (0, 0)),
                pl.BlockSpec((ND, 1), lambda i, pos: (0, 0)),
            ],
        ),
        out_shape=[
            jax.ShapeDtypeStruct((ND, D_MODEL), jnp.float32),
            jax.ShapeDtypeStruct((ND, 1), jnp.float32),
        ],
    )(pos.reshape(-1), xn, w_all)


def _ffn_krn(te_ref, nt_ref, xd_ref, wd_ref, w1_ref, w2_ref, yd_ref):
    t = pl.program_id(0)

    @pl.when(t < nt_ref[0])
    def _():
        h = jnp.dot(_bf(xd_ref[...]), _bf(w1_ref[0]),
                    preferred_element_type=jnp.float32)
        h = 0.5 * h * (1.0 + jax.lax.erf(h / np.sqrt(2.0).astype(np.float32)))
        yd_ref[...] = wd_ref[...] * jnp.dot(
            _bf(h), _bf(w2_ref[0]), preferred_element_type=jnp.float32,
        )


def _ffn(te, nt, xd, wd, w1, w2):
    ND = MAXT * T_DISP
    return pl.pallas_call(
        _ffn_krn,
        grid_spec=pltpu.PrefetchScalarGridSpec(
            num_scalar_prefetch=2,
            grid=(MAXT,),
            in_specs=[
                pl.BlockSpec(
                    (T_DISP, D_MODEL),
                    lambda t, te, nt: (jnp.where(t < nt[0], t, 0), 0),
                ),
                pl.BlockSpec(
                    (T_DISP, 1), lambda t, te, nt: (jnp.where(t < nt[0], t, 0), 0)
                ),
                pl.BlockSpec((1, D_MODEL, D_FF), lambda t, te, nt: (te[t], 0, 0)),
                pl.BlockSpec((1, D_FF, D_MODEL), lambda t, te, nt: (te[t], 0, 0)),
            ],
            out_specs=pl.BlockSpec(
                (T_DISP, D_MODEL),
                lambda t, te, nt: (jnp.where(t < nt[0], t, MAXT - 1), 0),
            ),
        ),
        out_shape=jax.ShapeDtypeStruct((ND, D_MODEL), jnp.float32),
    )(te.reshape(-1), nt.reshape(-1), xd, wd, w1, w2)


def _comb_krn(pos_ref, x_ref, yd_ref, o_ref):
    i = pl.program_id(0)
    L = x_ref.shape[0] * pl.num_programs(0)
    base = i * ROW_BLK

    def body(j, _):
        p0 = pos_ref[base + j]
        p1 = pos_ref[L + base + j]
        o_ref[pl.ds(j, 1), :] = x_ref[pl.ds(j, 1), :] + (
            yd_ref[pl.ds(p0, 1), :] + yd_ref[pl.ds(p1, 1), :]
        )
        return 0

    jax.lax.fori_loop(0, ROW_BLK, body, 0, unroll=8)


def _combine(pos, x, yd, L):
    ND = MAXT * T_DISP
    nblk = L // ROW_BLK
    return pl.pallas_call(
        _comb_krn,
        grid_spec=pltpu.PrefetchScalarGridSpec(
            num_scalar_prefetch=1,
            grid=(nblk,),
            in_specs=[
                pl.BlockSpec((ROW_BLK, D_MODEL), lambda i, pos: (i, 0)),
                pl.BlockSpec((ND, D_MODEL), lambda i, pos: (0, 0)),
            ],
            out_specs=pl.BlockSpec((ROW_BLK, D_MODEL), lambda i, pos: (i, 0)),
        ),
        out_shape=jax.ShapeDtypeStruct((L, D_MODEL), jnp.float32),
    )(pos.reshape(-1), x, yd)


def _moe_sparse(x, g, b, gate_w, w1, w2, L):
    xn, e1, e2, wa, wb = _router(x, g, b, gate_w, L)
    e_all = jnp.concatenate([e1, e2], axis=0)
    w_all = jnp.concatenate([wa, wb], axis=0)
    rank, cnt = _ranks(e_all, 2 * L)
    pos, te, nt = _positions(cnt, rank, e_all, 2 * L)
    xd, wd = _dispatch(pos, xn, w_all, L)
    yd = _ffn(te, nt, xd, wd, w1, w2)
    pos0, pos1 = pos[:L], pos[L:]
    swap = e1 > e2
    pos_lo = jnp.where(swap, pos1, pos0)
    pos_hi = jnp.where(swap, pos0, pos1)
    pos_ord = jnp.concatenate([pos_lo, pos_hi], axis=0)
    return _combine(pos_ord, x, yd, L)


def _moe_krn(x_ref, g_ref, b_ref, gw_ref, w1_ref, w2_ref, o_ref, acc_ref):
    e = pl.program_id(0)
    i = pl.program_id(1)
    x = x_ref[...]
    xn = _ln(x, g_ref[0], b_ref[0])
    logits = jax.lax.dot_general(
        xn, gw_ref[...], (((1,), (1,)), ((), ())),
        preferred_element_type=jnp.float32, precision=HI,
    )
    ids8 = jax.lax.broadcasted_iota(jnp.int32, logits.shape, 1)
    m1 = jnp.max(logits, axis=-1, keepdims=True)
    i1 = jnp.min(jnp.where(logits == m1, ids8, N_EXPERTS), axis=-1, keepdims=True)
    masked = jnp.where(ids8 == i1, -jnp.inf, logits)
    m2 = jnp.max(masked, axis=-1, keepdims=True)
    i2 = jnp.min(jnp.where(masked == m2, ids8, N_EXPERTS), axis=-1, keepdims=True)
    s = jnp.exp(m2 - m1)
    wa = 1.0 / (1.0 + s)
    wb = 1.0 - wa
    we = jnp.where(i1 == e, wa, 0.0) + jnp.where(i2 == e, wb, 0.0)

    h = jnp.dot(_bf(xn), _bf(w1_ref[0]), preferred_element_type=jnp.float32)
    h = 0.5 * h * (1.0 + jax.lax.erf(h / np.sqrt(2.0).astype(np.float32)))
    oe = jnp.dot(_bf(h), _bf(w2_ref[0]), preferred_element_type=jnp.float32)

    base = i * ROW_BLK

    @pl.when(e == 0)
    def _():
        acc_ref[pl.ds(base, ROW_BLK), :] = x

    acc_ref[pl.ds(base, ROW_BLK), :] += we * oe

    @pl.when(e == N_EXPERTS - 1)
    def _():
        o_ref[...] = acc_ref[pl.ds(base, ROW_BLK), :]


def _moe(x, g, b, gate_w, w1, w2, L):
    nblk = L // ROW_BLK
    return pl.pallas_call(
        _moe_krn,
        grid=(N_EXPERTS, nblk),
        in_specs=[
            pl.BlockSpec((ROW_BLK, D_MODEL), lambda e, i: (i, 0)),
            pl.BlockSpec((1, D_MODEL), lambda e, i: (0, 0)),
            pl.BlockSpec((1, D_MODEL), lambda e, i: (0, 0)),
            pl.BlockSpec((N_EXPERTS, D_MODEL), lambda e, i: (0, 0)),
            pl.BlockSpec((1, D_MODEL, D_FF), lambda e, i: (e, 0, 0)),
            pl.BlockSpec((1, D_FF, D_MODEL), lambda e, i: (e, 0, 0)),
        ],
        out_specs=pl.BlockSpec((ROW_BLK, D_MODEL), lambda e, i: (i, 0)),
        out_shape=jax.ShapeDtypeStruct((L, D_MODEL), jnp.float32),
        scratch_shapes=[pltpu.VMEM((L, D_MODEL), jnp.float32)],
    )(x, g.reshape(1, -1), b.reshape(1, -1), gate_w, w1, w2)


def _head_krn(x_ref, g_ref, b_ref, w_ref, o_ref):
    xn = _ln(x_ref[...], g_ref[0], b_ref[0])
    o_ref[...] = jax.lax.dot_general(
        _bf(xn), _bf(w_ref[...]), (((1,), (1,)), ((), ())),
        preferred_element_type=jnp.float32,
    )


def _head(x, g, b, w, L):
    vblk = 512
    return pl.pallas_call(
        _head_krn,
        grid=(VOCAB // vblk,),
        in_specs=[
            pl.BlockSpec((L, D_MODEL), lambda j: (0, 0)),
            pl.BlockSpec((1, D_MODEL), lambda j: (0, 0)),
            pl.BlockSpec((1, D_MODEL), lambda j: (0, 0)),
            pl.BlockSpec((vblk, D_MODEL), lambda j: (j, 0)),
        ],
        out_specs=pl.BlockSpec((L, vblk), lambda j: (0, j)),
        out_shape=jax.ShapeDtypeStruct((L, VOCAB), jnp.float32),
    )(x, g.reshape(1, -1), b.reshape(1, -1), w)


def kernel(input_ids, params):
    B, L = input_ids.shape
    ids = input_ids.reshape(-1)

    x = _embed(ids, params['tok_emb'], params['pos_emb'][:L], L)

    for lp in params['layers']:
        qkv = _ln_mm(x, lp['ln1_g'], lp['ln1_b'], lp['in_w'], lp['in_b'], L)
        q, k, v = jnp.split(qkv, 3, axis=-1)

        def sh(t):
            return t.reshape(L, N_HEADS, HEAD_DIM).transpose(1, 0, 2)

        o = _attention(sh(q), sh(k), sh(v), L)
        o = o.transpose(1, 0, 2).reshape(L, D_MODEL)
        x = _mm_res(o, lp['out_w'], lp['out_b'], x, L)

        x = _moe_sparse(
            x, lp['ln2_g'], lp['ln2_b'], lp['gate_w'], lp['w1'], lp['w2'], L
        )

    logits = _head(x, params['lnf_g'], params['lnf_b'], params['head_w'], L)
    return logits.reshape(B, L, VOCAB)
